# baseline (speedup 1.0000x reference)
import functools

import jax
import jax.numpy as jnp
from jax import lax
from jax.experimental import pallas as pl
from jax.experimental.pallas import tpu as pltpu
from jax.experimental.pallas import tpu_sc as plsc

N = 10000
E = 160000
NC, NS = 2, 16
NW = NC * NS
NR = 320
NPAD = NW * NR
RP_PAD = NPAD + 16
K1 = 64
K2 = 32
K3 = 64
EW = E // NW
NCH3 = -(-EW // K3)
EPAD = E + 1088

_MESH = plsc.VectorSubcoreMesh(core_axis_name="c", subcore_axis_name="s",
                               num_cores=NC, num_subcores=NS)
_SC_PARAMS = pltpu.CompilerParams(needs_layout_passes=False)
_IOTA = functools.partial(jnp.arange, dtype=jnp.int32)


def _wid():
    return lax.axis_index("s") * NC + lax.axis_index("c")


def _rsqrt16(x):
    i = plsc.bitcast(x, jnp.int32)
    y = plsc.bitcast(jnp.int32(0x5F3759DF) - (i >> 1), jnp.float32)
    for _ in range(3):
        y = y * (1.5 - 0.5 * x * y * y)
    return jnp.where(x > 0.0, y, 0.0)


def _make_sweep1(D, sig_idx):
    GC = 16 if D == 256 else 32
    NG = 8
    SK = GC * NG

    @functools.partial(
        pl.kernel,
        out_type=(jax.ShapeDtypeStruct((EPAD,), jnp.float32),
                  jax.ShapeDtypeStruct((NPAD,), jnp.float32)),
        mesh=_MESH,
        compiler_params=_SC_PARAMS,
        scratch_types=[
            pltpu.VMEM((16,), jnp.float32),
            pltpu.VMEM((336,), jnp.int32),
            pltpu.VMEM((SK + 16,), jnp.int32),
            pltpu.VMEM((SK + 16,), jnp.int32),
            pltpu.VMEM((GC, D), jnp.float32),
            pltpu.VMEM((GC, D), jnp.float32),
            pltpu.VMEM((GC, D), jnp.float32),
            pltpu.VMEM((GC, D), jnp.float32),
            pltpu.VMEM((GC * 16,), jnp.float32),
            pltpu.VMEM((SK,), jnp.float32),
            pltpu.VMEM((NR,), jnp.float32),
            pltpu.SemaphoreType.DMA,
            pltpu.SemaphoreType.DMA,
            pltpu.SemaphoreType.DMA,
            pltpu.SemaphoreType.DMA,
        ],
        name=f"gnn_sweep1_d{D}_s{sig_idx}",
    )
    def sweep1(feat, srow, scol, row_ptr, sig16, val_o, deg_o,
               sig_v, rp_v, sr_s, sc_s, a0, a1, b0, b1, d2p_v, val_s, deg_v,
               sa0, sa1, sb0, sb1):
        w = _wid()
        lo = w * NR
        hi = lo + NR
        pltpu.sync_copy(sig16, sig_v)
        sv = sig_v[pl.ds(0, 16)]
        inv = (-1.0 / (sv * sv))[sig_idx]
        pltpu.sync_copy(row_ptr.at[pl.ds(lo, 336)], rp_v)
        e_lo = rp_v[pl.ds(0, 16)][0]
        e_hi = rp_v[pl.ds(NR, 16)][0]
        e_base = (e_lo // 8) * 8
        nsc = (e_hi - e_base + SK - 1) // SK
        zeros = jnp.zeros((16,), jnp.float32)
        iota = _IOTA(16)
        for i in range(NR // 16):
            deg_v[pl.ds(i * 16, 16)] = zeros
        bufs = [(a0, b0, sa0, sb0), (a1, b1, sa1, sb1)]

        def sc_body(scj, _):
            s0 = e_base + scj * SK
            pltpu.sync_copy(srow.at[pl.ds(s0, SK + 16)], sr_s)
            pltpu.sync_copy(scol.at[pl.ds(s0, SK + 16)], sc_s)

            def clamp_body(g, _):
                sl = pl.ds(g * 16, 16)
                sr_s[sl] = jnp.minimum(jnp.maximum(sr_s[sl], 0), N - 1)
                sc_s[sl] = jnp.minimum(jnp.maximum(sc_s[sl], 0), N - 1)
                return 0

            lax.fori_loop(0, (SK + 16) // 16, clamp_body, 0)

            def issue(c, bf):
                a_v, b_v, sema, semb = bf
                pltpu.async_copy(feat.at[sr_s.at[pl.ds(c * GC, GC)]],
                                 a_v, sema)
                pltpu.async_copy(feat.at[sc_s.at[pl.ds(c * GC, GC)]],
                                 b_v, semb)

            def wait(c, bf):
                a_v, b_v, sema, semb = bf
                pltpu.make_async_copy(feat.at[sr_s.at[pl.ds(c * GC, GC)]],
                                      a_v, sema).wait()
                pltpu.make_async_copy(feat.at[sc_s.at[pl.ds(c * GC, GC)]],
                                      b_v, semb).wait()

            issue(0, bufs[0])
            for c in range(NG):
                if c + 1 < NG:
                    issue(c + 1, bufs[(c + 1) % 2])
                wait(c, bufs[c % 2])
                a_v, b_v = bufs[c % 2][0], bufs[c % 2][1]

                def edge_body(k, _):
                    acc = zeros
                    for d in range(D // 16):
                        av = a_v[k, pl.ds(d * 16, 16)]
                        bv = b_v[k, pl.ds(d * 16, 16)]
                        df = av - bv
                        acc = acc + df * df
                    d2p_v[pl.ds(k * 16, 16)] = acc
                    return 0

                lax.fori_loop(0, GC, edge_body, 0)

                for kk in range(GC // 16):
                    kv = kk * 16 + iota
                    d2 = zeros
                    for jj in range(16):
                        d2 = d2 + plsc.load_gather(d2p_v, [kv * 16 + jj])
                    valv = jnp.exp(d2 * inv)
                    off = c * GC + kk * 16
                    val_s[pl.ds(off, 16)] = valv
                    rowv = sr_s[pl.ds(off, 16)]
                    ev = s0 + off + iota
                    m = (rowv >= lo) & (rowv < hi) & (ev < E)
                    plsc.addupdate_scatter(deg_v, [rowv - lo], valv, mask=m)
            pltpu.sync_copy(val_s, val_o.at[pl.ds(s0, SK)])
            return 0

        lax.fori_loop(0, nsc, sc_body, 0)
        for i in range(NR // 16):
            sl = pl.ds(i * 16, 16)
            deg_v[sl] = _rsqrt16(deg_v[sl])
        pltpu.sync_copy(deg_v, deg_o.at[pl.ds(lo, NR)])

    return sweep1


def _make_sweep2(D):
    GC = 16 if D == 256 else 32
    NG = 8
    SK = GC * NG

    @functools.partial(
        pl.kernel,
        out_type=jax.ShapeDtypeStruct((NPAD, D), jnp.float32),
        mesh=_MESH,
        compiler_params=_SC_PARAMS,
        scratch_types=[
            pltpu.VMEM((336,), jnp.int32),
            pltpu.VMEM((SK + 16,), jnp.int32),
            pltpu.VMEM((SK + 16,), jnp.int32),
            pltpu.VMEM((SK + 16,), jnp.float32),
            pltpu.VMEM((GC + 16,), jnp.float32),
            pltpu.VMEM((GC + 16,), jnp.int32),
            pltpu.VMEM((GC, D), jnp.float32),
            pltpu.VMEM((GC, D), jnp.float32),
            pltpu.VMEM((NPAD,), jnp.float32),
            pltpu.VMEM((NR, D), jnp.float32),
            pltpu.SemaphoreType.DMA,
            pltpu.SemaphoreType.DMA,
        ],
        name=f"gnn_sweep2_d{D}",
    )
    def sweep2(feat, srow, scol, row_ptr, val, dis, out_o,
               rp_v, sr_s, sc_s, val_s, w_v, m_v, b0, b1, dis_v, out_v,
               sb0, sb1):
        w = _wid()
        lo = w * NR
        hi = lo + NR
        pltpu.sync_copy(row_ptr.at[pl.ds(lo, 336)], rp_v)
        e_lo = rp_v[pl.ds(0, 16)][0]
        e_hi = rp_v[pl.ds(NR, 16)][0]
        e_base = (e_lo // 8) * 8
        nsc = (e_hi - e_base + SK - 1) // SK

        pltpu.sync_copy(dis, dis_v)

        zeros = jnp.zeros((16,), jnp.float32)
        iota = _IOTA(16)

        def zrow_body(r, _):
            for d in range(D // 16):
                out_v[r, pl.ds(d * 16, 16)] = zeros
            return 0

        lax.fori_loop(0, NR, zrow_body, 0)
        bufs = [(b0, sb0), (b1, sb1)]

        def sc_body(scj, _):
            s0 = e_base + scj * SK
            pltpu.sync_copy(srow.at[pl.ds(s0, SK + 16)], sr_s)
            pltpu.sync_copy(scol.at[pl.ds(s0, SK + 16)], sc_s)
            pltpu.sync_copy(val.at[pl.ds(s0, SK + 16)], val_s)

            def clamp_body(g, _):
                sl = pl.ds(g * 16, 16)
                sr_s[sl] = jnp.minimum(jnp.maximum(sr_s[sl], 0), N - 1)
                sc_s[sl] = jnp.minimum(jnp.maximum(sc_s[sl], 0), N - 1)
                return 0

            lax.fori_loop(0, (SK + 16) // 16, clamp_body, 0)

            def issue(c, bf):
                pltpu.async_copy(feat.at[sc_s.at[pl.ds(c * GC, GC)]],
                                 bf[0], bf[1])

            def wait(c, bf):
                pltpu.make_async_copy(feat.at[sc_s.at[pl.ds(c * GC, GC)]],
                                      bf[0], bf[1]).wait()

            issue(0, bufs[0])
            for c in range(NG):
                if c + 1 < NG:
                    issue(c + 1, bufs[(c + 1) % 2])
                wait(c, bufs[c % 2])
                b_v = bufs[c % 2][0]
                for kk in range(GC // 16):
                    off = c * GC + kk * 16
                    sl = pl.ds(off, 16)
                    rowv = sr_s[sl]
                    colv = sc_s[sl]
                    disr = plsc.load_gather(dis_v, [rowv])
                    disc = plsc.load_gather(dis_v, [colv])
                    ev = s0 + off + iota
                    m = (rowv >= lo) & (rowv < hi) & (ev < E)
                    w_v[pl.ds(kk * 16, 16)] = jnp.where(
                        m, val_s[sl] * disr * disc, 0.0)
                    m_v[pl.ds(kk * 16, 16)] = jnp.minimum(
                        jnp.maximum(rowv - lo, 0), NR - 1)

                def edge_body(k, _):
                    rloc = m_v[pl.ds(k, 16)][0]
                    we = w_v[pl.ds(k, 16)][0]
                    for d in range(D // 16):
                        sl = pl.ds(d * 16, 16)
                        out_v[rloc, sl] = out_v[rloc, sl] + we * b_v[k, sl]
                    return 0

                lax.fori_loop(0, GC, edge_body, 0)
            return 0

        lax.fori_loop(0, nsc, sc_body, 0)
        pltpu.sync_copy(out_v, out_o.at[pl.ds(lo, NR)])

    return sweep2


def _make_hist():
    @functools.partial(
        pl.kernel,
        out_type=jax.ShapeDtypeStruct((NW, NPAD), jnp.int32),
        mesh=_MESH,
        compiler_params=_SC_PARAMS,
        scratch_types=[
            pltpu.VMEM((K3,), jnp.int32),
            pltpu.VMEM((NPAD,), jnp.int32),
        ],
        name="gnn_hist",
    )
    def hist(row_pad, hist_o, rw_v, hist_v):
        w = _wid()
        e_lo = w * EW
        e_hi = e_lo + EW
        zi = jnp.zeros((16,), jnp.int32)
        ones = jnp.ones((16,), jnp.int32)

        def zbody(i, _):
            hist_v[pl.ds(i * 16, 16)] = zi
            return 0

        lax.fori_loop(0, NPAD // 16, zbody, 0)

        def chunk_body(j, _):
            e0 = e_lo + j * K3
            pltpu.sync_copy(row_pad.at[pl.ds(e0, K3)], rw_v)
            for kk in range(K3 // 16):
                rowv = rw_v[pl.ds(kk * 16, 16)]
                ev = e0 + kk * 16 + _IOTA(16)
                plsc.addupdate_scatter(hist_v, [rowv], ones, mask=ev < e_hi)
            return 0

        lax.fori_loop(0, NCH3, chunk_body, 0)
        pltpu.sync_copy(hist_v, hist_o.at[w])

    return hist


def _make_sort():
    @functools.partial(
        pl.kernel,
        out_type=(jax.ShapeDtypeStruct((EPAD,), jnp.int32),
                  jax.ShapeDtypeStruct((EPAD,), jnp.int32),
                  jax.ShapeDtypeStruct((RP_PAD,), jnp.int32)),
        mesh=_MESH,
        compiler_params=_SC_PARAMS,
        scratch_types=[
            pltpu.VMEM((NPAD,), jnp.int32),
            pltpu.VMEM((NPAD,), jnp.int32),
            pltpu.VMEM((NPAD,), jnp.int32),
            pltpu.VMEM((K3,), jnp.int32),
            pltpu.VMEM((K3,), jnp.int32),
            pltpu.VMEM((K3,), jnp.int32),
            pltpu.VMEM((K3,), jnp.int32),
            pltpu.VMEM((K3,), jnp.int32),
            pltpu.VMEM((K3,), jnp.int32),
            pltpu.VMEM((16,), jnp.int32),
            pltpu.SemaphoreType.DMA,
            pltpu.SemaphoreType.DMA,
            pltpu.SemaphoreType.DMA,
            pltpu.SemaphoreType.DMA,
        ],
        name="gnn_sort",
    )
    def csort(shist, row_pad, col_pad, srow_o, scol_o, rp_o,
              tmp_v, pre_v, tot_v, rw_v, cl_v, pos_v, rw2_v, cl2_v, pos2_v,
              c16_v, sem_s, sem_c, sem_s2, sem_c2):
        w = _wid()
        lo = w * NR
        e_lo = w * EW
        e_hi = e_lo + EW
        zi = jnp.zeros((16,), jnp.int32)
        ones = jnp.ones((16,), jnp.int32)
        iota = _IOTA(16)

        def zbody(i, _):
            pre_v[pl.ds(i * 16, 16)] = zi
            tot_v[pl.ds(i * 16, 16)] = zi
            return 0

        lax.fori_loop(0, NPAD // 16, zbody, 0)

        for wp in range(NW):
            pltpu.sync_copy(shist.at[wp], tmp_v)
            sel = jnp.where(wp < w, 1, 0)

            def abody(i, _):
                sl = pl.ds(i * 16, 16)
                t = tmp_v[sl]
                tot_v[sl] = tot_v[sl] + t
                pre_v[sl] = pre_v[sl] + t * sel
                return 0

            lax.fori_loop(0, NPAD // 16, abody, 0)

        def sbody(i, carry):
            sl = pl.ds(i * 16, 16)
            seg = tot_v[sl]
            c = plsc.cumsum(seg)
            tot_v[sl] = c - seg + carry
            return carry + c[15]

        lax.fori_loop(0, NPAD // 16, sbody, jnp.int32(0))

        def obody(i, _):
            sl = pl.ds(i * 16, 16)
            pre_v[sl] = pre_v[sl] + tot_v[sl]
            return 0

        lax.fori_loop(0, NPAD // 16, obody, 0)

        pltpu.sync_copy(tot_v.at[pl.ds(lo, NR)], rp_o.at[pl.ds(lo, NR)])

        @pl.when(w == 0)
        def _():
            c16_v[pl.ds(0, 16)] = jnp.full((16,), E, jnp.int32)
            pltpu.sync_copy(c16_v, rp_o.at[pl.ds(NPAD, 16)])
            for q in range(4):
                pos_v[pl.ds(q * 16, 16)] = zi
            pltpu.sync_copy(pos_v, srow_o.at[pl.ds(E, K3)])
            pltpu.sync_copy(pos_v, scol_o.at[pl.ds(E, K3)])

        def compute_chunk(j, rw, cl, pos):
            e0 = e_lo + j * K3
            pltpu.sync_copy(row_pad.at[pl.ds(e0, K3)], rw)
            pltpu.sync_copy(col_pad.at[pl.ds(e0, K3)], cl)
            for kk in range(K3 // 16):
                sl = pl.ds(kk * 16, 16)
                rowv = rw[sl]
                rank = zi
                for s in range(1, 16):
                    pm = iota >= s
                    prv = plsc.load_gather(rw, [kk * 16 + iota - s],
                                           mask=pm)
                    rank = rank + jnp.where(pm & (prv == rowv), 1, 0)
                base = plsc.load_gather(pre_v, [rowv])
                ev = e0 + kk * 16 + iota
                valid = ev < e_hi
                dump = E + K3 + w * 16 + iota
                pos[sl] = jnp.where(valid, base + rank, dump)
                plsc.addupdate_scatter(pre_v, [rowv], ones, mask=valid)

        bufs = [(rw_v, cl_v, pos_v, sem_s, sem_c),
                (rw2_v, cl2_v, pos2_v, sem_s2, sem_c2)]

        def pair_body(jp, _):
            for st, (rw, cl, pos, ss, sc) in enumerate(bufs):
                @pl.when(jp > 0)
                def _():
                    pltpu.make_async_copy(rw, srow_o.at[pos], ss).wait()
                    pltpu.make_async_copy(cl, scol_o.at[pos], sc).wait()

                compute_chunk(jp * 2 + st, rw, cl, pos)
                pltpu.async_copy(rw, srow_o.at[pos], ss)
                pltpu.async_copy(cl, scol_o.at[pos], sc)
            return 0

        lax.fori_loop(0, (NCH3 + 1) // 2, pair_body, 0)
        for rw, cl, pos, ss, sc in bufs:
            pltpu.make_async_copy(rw, srow_o.at[pos], ss).wait()
            pltpu.make_async_copy(cl, scol_o.at[pos], sc).wait()

    return csort


def _make_ybuild():
    @functools.partial(
        pl.kernel,
        out_type=jax.ShapeDtypeStruct((NPAD, 128), jnp.float32),
        mesh=_MESH,
        compiler_params=_SC_PARAMS,
        scratch_types=[
            pltpu.VMEM((1024,), jnp.int32),
            pltpu.VMEM((NR + 16,), jnp.float32),
            pltpu.VMEM((NR, 128), jnp.float32),
        ],
        name="gnn_ybuild",
    )
    def ybuild(y_pad, roots, y_o, roots_v, fac_v, y_v):
        w = _wid()
        lo = w * NR
        hi = lo + NR
        pltpu.sync_copy(roots, roots_v)
        pltpu.sync_copy(y_pad.at[pl.ds(lo, NR)], y_v)
        ones = jnp.ones((16,), jnp.float32)
        zeros = jnp.zeros((16,), jnp.float32)
        for i in range((NR + 16) // 16):
            fac_v[pl.ds(i * 16, 16)] = ones
        for kk in range(1024 // 16):
            rv = roots_v[pl.ds(kk * 16, 16)]
            m = (rv >= lo) & (rv < hi)
            plsc.store_scatter(fac_v, [rv - lo], zeros, mask=m)

        def row_body(r, _):
            f = fac_v[pl.ds(r, 16)][0]
            for d in range(128 // 16):
                sl = pl.ds(d * 16, 16)
                y_v[r, sl] = y_v[r, sl] * f
            return 0

        lax.fori_loop(0, NR, row_body, 0)
        pltpu.sync_copy(y_v, y_o.at[pl.ds(lo, NR)])

    return ybuild


def _make_final_gather(dims):
    R = 1024
    RPW = R // NW

    @functools.partial(
        pl.kernel,
        out_type=tuple(jax.ShapeDtypeStruct((R, d), jnp.float32)
                       for d in dims),
        mesh=_MESH,
        compiler_params=_SC_PARAMS,
        scratch_types=[
            pltpu.VMEM((RPW,), jnp.int32),
            pltpu.VMEM((RPW, 256), jnp.float32),
            pltpu.VMEM((RPW, 128), jnp.float32),
            pltpu.SemaphoreType.DMA,
        ],
        name="gnn_final_gather",
    )
    def fgather(*args):
        srcs = args[:len(dims)]
        roots = args[len(dims)]
        outs = args[len(dims) + 1:len(dims) + 1 + len(dims)]
        ids_v, big_v, small_v, sem = args[len(dims) + 1 + len(dims):]
        w = _wid()
        r0 = w * RPW
        pltpu.sync_copy(roots.at[pl.ds(r0, RPW)], ids_v)
        for s, d, o in zip(srcs, dims, outs):
            buf = big_v if d == 256 else small_v
            pltpu.async_copy(s.at[ids_v], buf, sem).wait()
            pltpu.sync_copy(buf, o.at[pl.ds(r0, RPW)])

    return fgather


def _tc_matmul(parts, ws):
    n = len(parts)

    def body(*refs):
        o_ref = refs[-1]
        acc = jnp.zeros(o_ref.shape, jnp.float32)
        for a_ref, w_ref in zip(refs[:n], refs[n:-1]):
            acc = acc + jnp.dot(a_ref[...], w_ref[...],
                                preferred_element_type=jnp.float32)
        o_ref[...] = acc

    return pl.pallas_call(
        body,
        out_shape=jax.ShapeDtypeStruct((parts[0].shape[0], ws[0].shape[1]),
                                       jnp.float32),
    )(*parts, *ws)


def kernel(x, y_one_hot, W, sigmas, row, col, root_n_id):
    row_pad = jnp.pad(row, (0, EPAD - E))
    col_pad = jnp.pad(col, (0, EPAD - E))
    shist = _make_hist()(row_pad)
    srow, scol, row_ptr = _make_sort()(shist, row_pad, col_pad)
    sig16 = jnp.pad(sigmas, (0, 16 - sigmas.shape[0]))
    y_pad = jnp.pad(y_one_hot, ((0, NPAD - N), (0, 128 - 64)))

    feats = [x]
    feat = x
    for i in range(3):
        val, dis = _make_sweep1(256, i)(feat, srow, scol, row_ptr, sig16)
        feat = _make_sweep2(256)(feat, srow, scol, row_ptr, val, dis)
        feats.append(feat)

    g = _make_ybuild()(y_pad, root_n_id)
    for i in range(2):
        val, dis = _make_sweep1(128, 3 + i)(g, srow, scol, row_ptr, sig16)
        g = _make_sweep2(128)(g, srow, scol, row_ptr, val, dis)
        feats.append(g)

    parts = _make_final_gather((256, 256, 256, 256, 128, 128))(
        *feats, root_n_id)
    ws = [W[i * 256:(i + 1) * 256] for i in range(4)]
    ws.append(jnp.pad(W[1024:1088], ((0, 64), (0, 0))))
    ws.append(jnp.pad(W[1088:1152], ((0, 64), (0, 0))))
    return _tc_matmul(parts, ws)


# speedup vs baseline: 1.4090x; 1.4090x over previous
import functools

import jax
import jax.numpy as jnp
from jax import lax
from jax.experimental import pallas as pl
from jax.experimental.pallas import tpu as pltpu
from jax.experimental.pallas import tpu_sc as plsc

N = 10000
E = 160000
NC, NS = 2, 16
NW = NC * NS
NR = 320
NPAD = NW * NR
RP_PAD = NPAD + 16
K1 = 64
K2 = 32
K3 = 64
EW = E // NW
NCH3 = -(-EW // K3)
EPAD = E + 1088

_MESH = plsc.VectorSubcoreMesh(core_axis_name="c", subcore_axis_name="s",
                               num_cores=NC, num_subcores=NS)
_SC_PARAMS = pltpu.CompilerParams(needs_layout_passes=False)
_IOTA = functools.partial(jnp.arange, dtype=jnp.int32)


def _wid():
    return lax.axis_index("s") * NC + lax.axis_index("c")


def _rsqrt16(x):
    i = plsc.bitcast(x, jnp.int32)
    y = plsc.bitcast(jnp.int32(0x5F3759DF) - (i >> 1), jnp.float32)
    for _ in range(3):
        y = y * (1.5 - 0.5 * x * y * y)
    return jnp.where(x > 0.0, y, 0.0)


def _make_sweep1(D, sig_idx):
    GC = 16 if D == 256 else 32
    NG = 8
    SK = GC * NG

    @functools.partial(
        pl.kernel,
        out_type=(jax.ShapeDtypeStruct((EPAD,), jnp.float32),
                  jax.ShapeDtypeStruct((NPAD,), jnp.float32)),
        mesh=_MESH,
        compiler_params=_SC_PARAMS,
        scratch_types=[
            pltpu.VMEM((16,), jnp.float32),
            pltpu.VMEM((336,), jnp.int32),
            pltpu.VMEM((SK + 16,), jnp.int32),
            pltpu.VMEM((SK + 16,), jnp.int32),
            pltpu.VMEM((GC, D), jnp.float32),
            pltpu.VMEM((GC, D), jnp.float32),
            pltpu.VMEM((GC, D), jnp.float32),
            pltpu.VMEM((GC, D), jnp.float32),
            pltpu.VMEM((GC * 16,), jnp.float32),
            pltpu.VMEM((SK,), jnp.float32),
            pltpu.VMEM((NR,), jnp.float32),
            pltpu.SemaphoreType.DMA,
            pltpu.SemaphoreType.DMA,
            pltpu.SemaphoreType.DMA,
            pltpu.SemaphoreType.DMA,
        ],
        name=f"gnn_sweep1_d{D}_s{sig_idx}",
    )
    def sweep1(feat, srow, scol, row_ptr, sig16, val_o, deg_o,
               sig_v, rp_v, sr_s, sc_s, a0, a1, b0, b1, d2p_v, val_s, deg_v,
               sa0, sa1, sb0, sb1):
        w = _wid()
        lo = w * NR
        hi = lo + NR
        pltpu.sync_copy(sig16, sig_v)
        sv = sig_v[pl.ds(0, 16)]
        inv = (-1.0 / (sv * sv))[sig_idx]
        pltpu.sync_copy(row_ptr.at[pl.ds(lo, 336)], rp_v)
        e_lo = rp_v[pl.ds(0, 16)][0]
        e_hi = rp_v[pl.ds(NR, 16)][0]
        e_base = (e_lo // 8) * 8
        nsc = (e_hi - e_base + SK - 1) // SK
        zeros = jnp.zeros((16,), jnp.float32)
        iota = _IOTA(16)
        for i in range(NR // 16):
            deg_v[pl.ds(i * 16, 16)] = zeros
        bufs = [(a0, b0, sa0, sb0), (a1, b1, sa1, sb1)]

        def sc_body(scj, _):
            s0 = e_base + scj * SK
            pltpu.sync_copy(srow.at[pl.ds(s0, SK + 16)], sr_s)
            pltpu.sync_copy(scol.at[pl.ds(s0, SK + 16)], sc_s)

            def clamp_body(g, _):
                sl = pl.ds(g * 16, 16)
                sr_s[sl] = jnp.minimum(jnp.maximum(sr_s[sl], 0), N - 1)
                sc_s[sl] = jnp.minimum(jnp.maximum(sc_s[sl], 0), N - 1)
                return 0

            lax.fori_loop(0, (SK + 16) // 16, clamp_body, 0)

            def issue(c, bf):
                a_v, b_v, sema, semb = bf
                pltpu.async_copy(feat.at[sr_s.at[pl.ds(c * GC, GC)]],
                                 a_v, sema)
                pltpu.async_copy(feat.at[sc_s.at[pl.ds(c * GC, GC)]],
                                 b_v, semb)

            def wait(c, bf):
                a_v, b_v, sema, semb = bf
                pltpu.make_async_copy(feat.at[sr_s.at[pl.ds(c * GC, GC)]],
                                      a_v, sema).wait()
                pltpu.make_async_copy(feat.at[sc_s.at[pl.ds(c * GC, GC)]],
                                      b_v, semb).wait()

            issue(0, bufs[0])
            for c in range(NG):
                if c + 1 < NG:
                    issue(c + 1, bufs[(c + 1) % 2])
                wait(c, bufs[c % 2])
                a_v, b_v = bufs[c % 2][0], bufs[c % 2][1]

                def edge_body(k, _):
                    acc = zeros
                    for d in range(D // 16):
                        av = a_v[k, pl.ds(d * 16, 16)]
                        bv = b_v[k, pl.ds(d * 16, 16)]
                        df = av - bv
                        acc = acc + df * df
                    d2p_v[pl.ds(k * 16, 16)] = acc
                    return 0

                lax.fori_loop(0, GC, edge_body, 0)

                for kk in range(GC // 16):
                    kv = kk * 16 + iota
                    d2 = zeros
                    for jj in range(16):
                        d2 = d2 + plsc.load_gather(d2p_v, [kv * 16 + jj])
                    valv = jnp.exp(d2 * inv)
                    off = c * GC + kk * 16
                    val_s[pl.ds(off, 16)] = valv
                    rowv = sr_s[pl.ds(off, 16)]
                    ev = s0 + off + iota
                    m = (rowv >= lo) & (rowv < hi) & (ev < E)
                    plsc.addupdate_scatter(deg_v, [rowv - lo], valv, mask=m)
            pltpu.sync_copy(val_s, val_o.at[pl.ds(s0, SK)])
            return 0

        lax.fori_loop(0, nsc, sc_body, 0)
        for i in range(NR // 16):
            sl = pl.ds(i * 16, 16)
            deg_v[sl] = _rsqrt16(deg_v[sl])
        pltpu.sync_copy(deg_v, deg_o.at[pl.ds(lo, NR)])

    return sweep1


def _make_sweep2(D):
    GC = 16 if D == 256 else 32
    NG = 8
    SK = GC * NG

    @functools.partial(
        pl.kernel,
        out_type=jax.ShapeDtypeStruct((NPAD, D), jnp.float32),
        mesh=_MESH,
        compiler_params=_SC_PARAMS,
        scratch_types=[
            pltpu.VMEM((336,), jnp.int32),
            pltpu.VMEM((SK + 16,), jnp.int32),
            pltpu.VMEM((SK + 16,), jnp.int32),
            pltpu.VMEM((SK + 16,), jnp.float32),
            pltpu.VMEM((GC + 16,), jnp.float32),
            pltpu.VMEM((GC + 16,), jnp.int32),
            pltpu.VMEM((GC, D), jnp.float32),
            pltpu.VMEM((GC, D), jnp.float32),
            pltpu.VMEM((NPAD,), jnp.float32),
            pltpu.VMEM((NR, D), jnp.float32),
            pltpu.SemaphoreType.DMA,
            pltpu.SemaphoreType.DMA,
        ],
        name=f"gnn_sweep2_d{D}",
    )
    def sweep2(feat, srow, scol, row_ptr, val, dis, out_o,
               rp_v, sr_s, sc_s, val_s, w_v, m_v, b0, b1, dis_v, out_v,
               sb0, sb1):
        w = _wid()
        lo = w * NR
        hi = lo + NR
        pltpu.sync_copy(row_ptr.at[pl.ds(lo, 336)], rp_v)
        e_lo = rp_v[pl.ds(0, 16)][0]
        e_hi = rp_v[pl.ds(NR, 16)][0]
        e_base = (e_lo // 8) * 8
        nsc = (e_hi - e_base + SK - 1) // SK

        pltpu.sync_copy(dis, dis_v)

        zeros = jnp.zeros((16,), jnp.float32)
        iota = _IOTA(16)

        def zrow_body(r, _):
            for d in range(D // 16):
                out_v[r, pl.ds(d * 16, 16)] = zeros
            return 0

        lax.fori_loop(0, NR, zrow_body, 0)
        bufs = [(b0, sb0), (b1, sb1)]
        nacc = D // 16

        def sc_body(scj, carry):
            s0 = e_base + scj * SK
            pltpu.sync_copy(srow.at[pl.ds(s0, SK + 16)], sr_s)
            pltpu.sync_copy(scol.at[pl.ds(s0, SK + 16)], sc_s)
            pltpu.sync_copy(val.at[pl.ds(s0, SK + 16)], val_s)

            def clamp_body(g, _):
                sl = pl.ds(g * 16, 16)
                sr_s[sl] = jnp.minimum(jnp.maximum(sr_s[sl], 0), N - 1)
                sc_s[sl] = jnp.minimum(jnp.maximum(sc_s[sl], 0), N - 1)
                return 0

            lax.fori_loop(0, (SK + 16) // 16, clamp_body, 0)

            def issue(c, bf):
                pltpu.async_copy(feat.at[sc_s.at[pl.ds(c * GC, GC)]],
                                 bf[0], bf[1])

            def wait(c, bf):
                pltpu.make_async_copy(feat.at[sc_s.at[pl.ds(c * GC, GC)]],
                                      bf[0], bf[1]).wait()

            issue(0, bufs[0])
            for c in range(NG):
                if c + 1 < NG:
                    issue(c + 1, bufs[(c + 1) % 2])
                wait(c, bufs[c % 2])
                b_v = bufs[c % 2][0]
                for kk in range(GC // 16):
                    off = c * GC + kk * 16
                    sl = pl.ds(off, 16)
                    rowv = sr_s[sl]
                    colv = sc_s[sl]
                    disr = plsc.load_gather(dis_v, [rowv])
                    disc = plsc.load_gather(dis_v, [colv])
                    ev = s0 + off + iota
                    m = (rowv >= lo) & (rowv < hi) & (ev < E)
                    w_v[pl.ds(kk * 16, 16)] = jnp.where(
                        m, val_s[sl] * disr * disc, 0.0)
                    m_v[pl.ds(kk * 16, 16)] = jnp.minimum(
                        jnp.maximum(rowv - lo, 0), NR - 1)

                def edge_body(k, ec):
                    cur = ec[0]
                    acc = ec[1:]
                    rloc = m_v[pl.ds(k, 16)][0]
                    we = w_v[pl.ds(k, 16)][0]
                    changed = rloc != cur

                    @pl.when(changed)
                    def _():
                        for d in range(nacc):
                            sl = pl.ds(d * 16, 16)
                            out_v[cur, sl] = out_v[cur, sl] + acc[d]

                    keep = jnp.where(changed, 0.0, 1.0)
                    newacc = tuple(
                        acc[d] * keep + we * b_v[k, pl.ds(d * 16, 16)]
                        for d in range(nacc))
                    return (rloc,) + newacc

                carry = lax.fori_loop(0, GC, edge_body, carry)
            return carry

        zcarry = (jnp.int32(0),) + tuple(zeros for _ in range(nacc))
        fcarry = lax.fori_loop(0, nsc, sc_body, zcarry)
        fr = fcarry[0]
        for d in range(nacc):
            sl = pl.ds(d * 16, 16)
            out_v[fr, sl] = out_v[fr, sl] + fcarry[1 + d]
        pltpu.sync_copy(out_v, out_o.at[pl.ds(lo, NR)])

    return sweep2


def _make_hist():
    @functools.partial(
        pl.kernel,
        out_type=jax.ShapeDtypeStruct((NW, NPAD), jnp.int32),
        mesh=_MESH,
        compiler_params=_SC_PARAMS,
        scratch_types=[
            pltpu.VMEM((K3,), jnp.int32),
            pltpu.VMEM((NPAD,), jnp.int32),
        ],
        name="gnn_hist",
    )
    def hist(row_pad, hist_o, rw_v, hist_v):
        w = _wid()
        e_lo = w * EW
        e_hi = e_lo + EW
        zi = jnp.zeros((16,), jnp.int32)
        ones = jnp.ones((16,), jnp.int32)

        def zbody(i, _):
            hist_v[pl.ds(i * 16, 16)] = zi
            return 0

        lax.fori_loop(0, NPAD // 16, zbody, 0)

        def chunk_body(j, _):
            e0 = e_lo + j * K3
            pltpu.sync_copy(row_pad.at[pl.ds(e0, K3)], rw_v)
            for kk in range(K3 // 16):
                rowv = rw_v[pl.ds(kk * 16, 16)]
                ev = e0 + kk * 16 + _IOTA(16)
                plsc.addupdate_scatter(hist_v, [rowv], ones, mask=ev < e_hi)
            return 0

        lax.fori_loop(0, NCH3, chunk_body, 0)
        pltpu.sync_copy(hist_v, hist_o.at[w])

    return hist


def _make_sort():
    @functools.partial(
        pl.kernel,
        out_type=(jax.ShapeDtypeStruct((EPAD,), jnp.int32),
                  jax.ShapeDtypeStruct((EPAD,), jnp.int32),
                  jax.ShapeDtypeStruct((RP_PAD,), jnp.int32)),
        mesh=_MESH,
        compiler_params=_SC_PARAMS,
        scratch_types=[
            pltpu.VMEM((NPAD,), jnp.int32),
            pltpu.VMEM((NPAD,), jnp.int32),
            pltpu.VMEM((NPAD,), jnp.int32),
            pltpu.VMEM((K3,), jnp.int32),
            pltpu.VMEM((K3,), jnp.int32),
            pltpu.VMEM((K3,), jnp.int32),
            pltpu.VMEM((16,), jnp.int32),
            pltpu.SemaphoreType.DMA,
            pltpu.SemaphoreType.DMA,
        ],
        name="gnn_sort",
    )
    def csort(shist, row_pad, col_pad, srow_o, scol_o, rp_o,
              tmp_v, pre_v, tot_v, rw_v, cl_v, pos_v, c16_v, sem_s, sem_c):
        w = _wid()
        lo = w * NR
        e_lo = w * EW
        e_hi = e_lo + EW
        zi = jnp.zeros((16,), jnp.int32)
        ones = jnp.ones((16,), jnp.int32)
        iota = _IOTA(16)

        def zbody(i, _):
            pre_v[pl.ds(i * 16, 16)] = zi
            tot_v[pl.ds(i * 16, 16)] = zi
            return 0

        lax.fori_loop(0, NPAD // 16, zbody, 0)

        for wp in range(NW):
            pltpu.sync_copy(shist.at[wp], tmp_v)
            sel = jnp.where(wp < w, 1, 0)

            def abody(i, _):
                sl = pl.ds(i * 16, 16)
                t = tmp_v[sl]
                tot_v[sl] = tot_v[sl] + t
                pre_v[sl] = pre_v[sl] + t * sel
                return 0

            lax.fori_loop(0, NPAD // 16, abody, 0)

        def sbody(i, carry):
            sl = pl.ds(i * 16, 16)
            seg = tot_v[sl]
            c = plsc.cumsum(seg)
            tot_v[sl] = c - seg + carry
            return carry + c[15]

        lax.fori_loop(0, NPAD // 16, sbody, jnp.int32(0))

        def obody(i, _):
            sl = pl.ds(i * 16, 16)
            pre_v[sl] = pre_v[sl] + tot_v[sl]
            return 0

        lax.fori_loop(0, NPAD // 16, obody, 0)

        pltpu.sync_copy(tot_v.at[pl.ds(lo, NR)], rp_o.at[pl.ds(lo, NR)])

        @pl.when(w == 0)
        def _():
            c16_v[pl.ds(0, 16)] = jnp.full((16,), E, jnp.int32)
            pltpu.sync_copy(c16_v, rp_o.at[pl.ds(NPAD, 16)])
            for q in range(4):
                pos_v[pl.ds(q * 16, 16)] = zi
            pltpu.sync_copy(pos_v, srow_o.at[pl.ds(E, K3)])
            pltpu.sync_copy(pos_v, scol_o.at[pl.ds(E, K3)])

        def chunk_body(j, _):
            e0 = e_lo + j * K3
            pltpu.sync_copy(row_pad.at[pl.ds(e0, K3)], rw_v)
            pltpu.sync_copy(col_pad.at[pl.ds(e0, K3)], cl_v)
            for kk in range(K3 // 16):
                sl = pl.ds(kk * 16, 16)
                rowv = rw_v[sl]
                rank = zi
                for s in range(1, 16):
                    pm = iota >= s
                    prv = plsc.load_gather(rw_v, [kk * 16 + iota - s],
                                           mask=pm)
                    rank = rank + jnp.where(pm & (prv == rowv), 1, 0)
                base = plsc.load_gather(pre_v, [rowv])
                ev = e0 + kk * 16 + iota
                valid = ev < e_hi
                dump = E + K3 + w * 16 + iota
                pos_v[sl] = jnp.where(valid, base + rank, dump)
                plsc.addupdate_scatter(pre_v, [rowv], ones, mask=valid)
            cp_s = pltpu.async_copy(rw_v, srow_o.at[pos_v], sem_s)
            cp_c = pltpu.async_copy(cl_v, scol_o.at[pos_v], sem_c)
            cp_s.wait()
            cp_c.wait()
            return 0

        lax.fori_loop(0, NCH3, chunk_body, 0)

    return csort


def _make_ybuild():
    @functools.partial(
        pl.kernel,
        out_type=jax.ShapeDtypeStruct((NPAD, 128), jnp.float32),
        mesh=_MESH,
        compiler_params=_SC_PARAMS,
        scratch_types=[
            pltpu.VMEM((1024,), jnp.int32),
            pltpu.VMEM((NR + 16,), jnp.float32),
            pltpu.VMEM((NR, 128), jnp.float32),
        ],
        name="gnn_ybuild",
    )
    def ybuild(y_pad, roots, y_o, roots_v, fac_v, y_v):
        w = _wid()
        lo = w * NR
        hi = lo + NR
        pltpu.sync_copy(roots, roots_v)
        pltpu.sync_copy(y_pad.at[pl.ds(lo, NR)], y_v)
        ones = jnp.ones((16,), jnp.float32)
        zeros = jnp.zeros((16,), jnp.float32)
        for i in range((NR + 16) // 16):
            fac_v[pl.ds(i * 16, 16)] = ones
        for kk in range(1024 // 16):
            rv = roots_v[pl.ds(kk * 16, 16)]
            m = (rv >= lo) & (rv < hi)
            plsc.store_scatter(fac_v, [rv - lo], zeros, mask=m)

        def row_body(r, _):
            f = fac_v[pl.ds(r, 16)][0]
            for d in range(128 // 16):
                sl = pl.ds(d * 16, 16)
                y_v[r, sl] = y_v[r, sl] * f
            return 0

        lax.fori_loop(0, NR, row_body, 0)
        pltpu.sync_copy(y_v, y_o.at[pl.ds(lo, NR)])

    return ybuild


def _make_final_gather(dims):
    R = 1024
    RPW = R // NW

    @functools.partial(
        pl.kernel,
        out_type=tuple(jax.ShapeDtypeStruct((R, d), jnp.float32)
                       for d in dims),
        mesh=_MESH,
        compiler_params=_SC_PARAMS,
        scratch_types=[
            pltpu.VMEM((RPW,), jnp.int32),
            pltpu.VMEM((RPW, 256), jnp.float32),
            pltpu.VMEM((RPW, 128), jnp.float32),
            pltpu.SemaphoreType.DMA,
        ],
        name="gnn_final_gather",
    )
    def fgather(*args):
        srcs = args[:len(dims)]
        roots = args[len(dims)]
        outs = args[len(dims) + 1:len(dims) + 1 + len(dims)]
        ids_v, big_v, small_v, sem = args[len(dims) + 1 + len(dims):]
        w = _wid()
        r0 = w * RPW
        pltpu.sync_copy(roots.at[pl.ds(r0, RPW)], ids_v)
        for s, d, o in zip(srcs, dims, outs):
            buf = big_v if d == 256 else small_v
            pltpu.async_copy(s.at[ids_v], buf, sem).wait()
            pltpu.sync_copy(buf, o.at[pl.ds(r0, RPW)])

    return fgather


def _tc_matmul(parts, ws):
    n = len(parts)

    def body(*refs):
        o_ref = refs[-1]
        acc = jnp.zeros(o_ref.shape, jnp.float32)
        for a_ref, w_ref in zip(refs[:n], refs[n:-1]):
            acc = acc + jnp.dot(a_ref[...], w_ref[...],
                                preferred_element_type=jnp.float32)
        o_ref[...] = acc

    return pl.pallas_call(
        body,
        out_shape=jax.ShapeDtypeStruct((parts[0].shape[0], ws[0].shape[1]),
                                       jnp.float32),
    )(*parts, *ws)


def kernel(x, y_one_hot, W, sigmas, row, col, root_n_id):
    row_pad = jnp.pad(row, (0, EPAD - E))
    col_pad = jnp.pad(col, (0, EPAD - E))
    shist = _make_hist()(row_pad)
    srow, scol, row_ptr = _make_sort()(shist, row_pad, col_pad)
    sig16 = jnp.pad(sigmas, (0, 16 - sigmas.shape[0]))
    y_pad = jnp.pad(y_one_hot, ((0, NPAD - N), (0, 128 - 64)))

    feats = [x]
    feat = x
    for i in range(3):
        val, dis = _make_sweep1(256, i)(feat, srow, scol, row_ptr, sig16)
        feat = _make_sweep2(256)(feat, srow, scol, row_ptr, val, dis)
        feats.append(feat)

    g = _make_ybuild()(y_pad, root_n_id)
    for i in range(2):
        val, dis = _make_sweep1(128, 3 + i)(g, srow, scol, row_ptr, sig16)
        g = _make_sweep2(128)(g, srow, scol, row_ptr, val, dis)
        feats.append(g)

    parts = _make_final_gather((256, 256, 256, 256, 128, 128))(
        *feats, root_n_id)
    ws = [W[i * 256:(i + 1) * 256] for i in range(4)]
    ws.append(jnp.pad(W[1024:1088], ((0, 64), (0, 0))))
    ws.append(jnp.pad(W[1088:1152], ((0, 64), (0, 0))))
    return _tc_matmul(parts, ws)


# speedup vs baseline: 1.5889x; 1.1277x over previous
import functools

import jax
import jax.numpy as jnp
from jax import lax
from jax.experimental import pallas as pl
from jax.experimental.pallas import tpu as pltpu
from jax.experimental.pallas import tpu_sc as plsc

N = 10000
E = 160000
NC, NS = 2, 16
NW = NC * NS
NR = 320
NPAD = NW * NR
RP_PAD = NPAD + 16
K1 = 64
K2 = 32
K3 = 64
EW = E // NW
NCH3 = -(-EW // K3)
EPAD = E + 1088

_MESH = plsc.VectorSubcoreMesh(core_axis_name="c", subcore_axis_name="s",
                               num_cores=NC, num_subcores=NS)
_SC_PARAMS = pltpu.CompilerParams(needs_layout_passes=False)
_IOTA = functools.partial(jnp.arange, dtype=jnp.int32)


def _wid():
    return lax.axis_index("s") * NC + lax.axis_index("c")


def _rsqrt16(x):
    i = plsc.bitcast(x, jnp.int32)
    y = plsc.bitcast(jnp.int32(0x5F3759DF) - (i >> 1), jnp.float32)
    for _ in range(3):
        y = y * (1.5 - 0.5 * x * y * y)
    return jnp.where(x > 0.0, y, 0.0)


def _make_sweep1(D, sig_idx):
    GC = 16 if D == 256 else 32
    NG = 8
    SK = GC * NG

    @functools.partial(
        pl.kernel,
        out_type=(jax.ShapeDtypeStruct((EPAD,), jnp.float32),
                  jax.ShapeDtypeStruct((NPAD,), jnp.float32)),
        mesh=_MESH,
        compiler_params=_SC_PARAMS,
        scratch_types=[
            pltpu.VMEM((16,), jnp.float32),
            pltpu.VMEM((336,), jnp.int32),
            pltpu.VMEM((SK + 16,), jnp.int32),
            pltpu.VMEM((SK + 16,), jnp.int32),
            pltpu.VMEM((NR, D), jnp.float32),
            pltpu.VMEM((GC, D), jnp.float32),
            pltpu.VMEM((GC, D), jnp.float32),
            pltpu.VMEM((GC * 16,), jnp.float32),
            pltpu.VMEM((SK,), jnp.float32),
            pltpu.VMEM((NR,), jnp.float32),
            pltpu.SemaphoreType.DMA,
            pltpu.SemaphoreType.DMA,
        ],
        name=f"gnn_sweep1_d{D}_s{sig_idx}",
    )
    def sweep1(feat, srow, scol, row_ptr, sig16, val_o, deg_o,
               sig_v, rp_v, sr_s, sc_s, floc_v, b0, b1, d2p_v, val_s, deg_v,
               sb0, sb1):
        w = _wid()
        lo = w * NR
        hi = lo + NR
        pltpu.sync_copy(sig16, sig_v)
        sv = sig_v[pl.ds(0, 16)]
        inv = (-1.0 / (sv * sv))[sig_idx]
        pltpu.sync_copy(feat.at[pl.ds(lo, NR)], floc_v)
        pltpu.sync_copy(row_ptr.at[pl.ds(lo, 336)], rp_v)
        e_lo = rp_v[pl.ds(0, 16)][0]
        e_hi = rp_v[pl.ds(NR, 16)][0]
        e_base = (e_lo // 8) * 8
        nsc = (e_hi - e_base + SK - 1) // SK
        zeros = jnp.zeros((16,), jnp.float32)
        iota = _IOTA(16)
        for i in range(NR // 16):
            deg_v[pl.ds(i * 16, 16)] = zeros
        bufs = [(b0, sb0), (b1, sb1)]

        def sc_body(scj, _):
            s0 = e_base + scj * SK
            pltpu.sync_copy(srow.at[pl.ds(s0, SK + 16)], sr_s)
            pltpu.sync_copy(scol.at[pl.ds(s0, SK + 16)], sc_s)

            def clamp_body(g, _):
                sl = pl.ds(g * 16, 16)
                sr_s[sl] = jnp.minimum(jnp.maximum(sr_s[sl], 0), N - 1)
                sc_s[sl] = jnp.minimum(jnp.maximum(sc_s[sl], 0), N - 1)
                return 0

            lax.fori_loop(0, (SK + 16) // 16, clamp_body, 0)

            def issue(c, bf):
                pltpu.async_copy(feat.at[sc_s.at[pl.ds(c * GC, GC)]],
                                 bf[0], bf[1])

            def wait(c, bf):
                pltpu.make_async_copy(feat.at[sc_s.at[pl.ds(c * GC, GC)]],
                                      bf[0], bf[1]).wait()

            issue(0, bufs[0])
            for c in range(NG):
                if c + 1 < NG:
                    issue(c + 1, bufs[(c + 1) % 2])
                wait(c, bufs[c % 2])
                b_v = bufs[c % 2][0]

                def edge_body(k, _):
                    rv = sr_s[pl.ds(c * GC + k, 16)][0]
                    rl = jnp.minimum(jnp.maximum(rv - lo, 0), NR - 1)
                    acc = zeros
                    for d in range(D // 16):
                        av = floc_v[rl, pl.ds(d * 16, 16)]
                        bv = b_v[k, pl.ds(d * 16, 16)]
                        df = av - bv
                        acc = acc + df * df
                    d2p_v[pl.ds(k * 16, 16)] = acc
                    return 0

                lax.fori_loop(0, GC, edge_body, 0)

                for kk in range(GC // 16):
                    kv = kk * 16 + iota
                    d2 = zeros
                    for jj in range(16):
                        d2 = d2 + plsc.load_gather(d2p_v, [kv * 16 + jj])
                    valv = jnp.exp(d2 * inv)
                    off = c * GC + kk * 16
                    val_s[pl.ds(off, 16)] = valv
                    rowv = sr_s[pl.ds(off, 16)]
                    ev = s0 + off + iota
                    m = (rowv >= lo) & (rowv < hi) & (ev < E)
                    plsc.addupdate_scatter(deg_v, [rowv - lo], valv, mask=m)
            pltpu.sync_copy(val_s, val_o.at[pl.ds(s0, SK)])
            return 0

        lax.fori_loop(0, nsc, sc_body, 0)
        for i in range(NR // 16):
            sl = pl.ds(i * 16, 16)
            deg_v[sl] = _rsqrt16(deg_v[sl])
        pltpu.sync_copy(deg_v, deg_o.at[pl.ds(lo, NR)])

    return sweep1


def _make_sweep2(D):
    GC = 16 if D == 256 else 32
    NG = 8
    SK = GC * NG

    @functools.partial(
        pl.kernel,
        out_type=jax.ShapeDtypeStruct((NPAD, D), jnp.float32),
        mesh=_MESH,
        compiler_params=_SC_PARAMS,
        scratch_types=[
            pltpu.VMEM((336,), jnp.int32),
            pltpu.VMEM((SK + 16,), jnp.int32),
            pltpu.VMEM((SK + 16,), jnp.int32),
            pltpu.VMEM((SK + 16,), jnp.float32),
            pltpu.VMEM((GC + 16,), jnp.float32),
            pltpu.VMEM((GC + 16,), jnp.int32),
            pltpu.VMEM((GC, D), jnp.float32),
            pltpu.VMEM((GC, D), jnp.float32),
            pltpu.VMEM((NPAD,), jnp.float32),
            pltpu.VMEM((NR, D), jnp.float32),
            pltpu.SemaphoreType.DMA,
            pltpu.SemaphoreType.DMA,
        ],
        name=f"gnn_sweep2_d{D}",
    )
    def sweep2(feat, srow, scol, row_ptr, val, dis, out_o,
               rp_v, sr_s, sc_s, val_s, w_v, m_v, b0, b1, dis_v, out_v,
               sb0, sb1):
        w = _wid()
        lo = w * NR
        hi = lo + NR
        pltpu.sync_copy(row_ptr.at[pl.ds(lo, 336)], rp_v)
        e_lo = rp_v[pl.ds(0, 16)][0]
        e_hi = rp_v[pl.ds(NR, 16)][0]
        e_base = (e_lo // 8) * 8
        nsc = (e_hi - e_base + SK - 1) // SK

        pltpu.sync_copy(dis, dis_v)

        zeros = jnp.zeros((16,), jnp.float32)
        iota = _IOTA(16)

        def zrow_body(r, _):
            for d in range(D // 16):
                out_v[r, pl.ds(d * 16, 16)] = zeros
            return 0

        lax.fori_loop(0, NR, zrow_body, 0)
        bufs = [(b0, sb0), (b1, sb1)]
        nacc = D // 16

        def sc_body(scj, carry):
            s0 = e_base + scj * SK
            pltpu.sync_copy(srow.at[pl.ds(s0, SK + 16)], sr_s)
            pltpu.sync_copy(scol.at[pl.ds(s0, SK + 16)], sc_s)
            pltpu.sync_copy(val.at[pl.ds(s0, SK + 16)], val_s)

            def clamp_body(g, _):
                sl = pl.ds(g * 16, 16)
                sr_s[sl] = jnp.minimum(jnp.maximum(sr_s[sl], 0), N - 1)
                sc_s[sl] = jnp.minimum(jnp.maximum(sc_s[sl], 0), N - 1)
                return 0

            lax.fori_loop(0, (SK + 16) // 16, clamp_body, 0)

            def issue(c, bf):
                pltpu.async_copy(feat.at[sc_s.at[pl.ds(c * GC, GC)]],
                                 bf[0], bf[1])

            def wait(c, bf):
                pltpu.make_async_copy(feat.at[sc_s.at[pl.ds(c * GC, GC)]],
                                      bf[0], bf[1]).wait()

            issue(0, bufs[0])
            for c in range(NG):
                if c + 1 < NG:
                    issue(c + 1, bufs[(c + 1) % 2])
                wait(c, bufs[c % 2])
                b_v = bufs[c % 2][0]
                for kk in range(GC // 16):
                    off = c * GC + kk * 16
                    sl = pl.ds(off, 16)
                    rowv = sr_s[sl]
                    colv = sc_s[sl]
                    disr = plsc.load_gather(dis_v, [rowv])
                    disc = plsc.load_gather(dis_v, [colv])
                    ev = s0 + off + iota
                    m = (rowv >= lo) & (rowv < hi) & (ev < E)
                    w_v[pl.ds(kk * 16, 16)] = jnp.where(
                        m, val_s[sl] * disr * disc, 0.0)
                    m_v[pl.ds(kk * 16, 16)] = jnp.minimum(
                        jnp.maximum(rowv - lo, 0), NR - 1)

                def edge_body(k, ec):
                    cur = ec[0]
                    acc = ec[1:]
                    rloc = m_v[pl.ds(k, 16)][0]
                    we = w_v[pl.ds(k, 16)][0]
                    changed = rloc != cur

                    @pl.when(changed)
                    def _():
                        for d in range(nacc):
                            sl = pl.ds(d * 16, 16)
                            out_v[cur, sl] = out_v[cur, sl] + acc[d]

                    keep = jnp.where(changed, 0.0, 1.0)
                    newacc = tuple(
                        acc[d] * keep + we * b_v[k, pl.ds(d * 16, 16)]
                        for d in range(nacc))
                    return (rloc,) + newacc

                carry = lax.fori_loop(0, GC, edge_body, carry)
            return carry

        zcarry = (jnp.int32(0),) + tuple(zeros for _ in range(nacc))
        fcarry = lax.fori_loop(0, nsc, sc_body, zcarry)
        fr = fcarry[0]
        for d in range(nacc):
            sl = pl.ds(d * 16, 16)
            out_v[fr, sl] = out_v[fr, sl] + fcarry[1 + d]
        pltpu.sync_copy(out_v, out_o.at[pl.ds(lo, NR)])

    return sweep2


def _make_hist():
    @functools.partial(
        pl.kernel,
        out_type=jax.ShapeDtypeStruct((NW, NPAD), jnp.int32),
        mesh=_MESH,
        compiler_params=_SC_PARAMS,
        scratch_types=[
            pltpu.VMEM((K3,), jnp.int32),
            pltpu.VMEM((NPAD,), jnp.int32),
        ],
        name="gnn_hist",
    )
    def hist(row_pad, hist_o, rw_v, hist_v):
        w = _wid()
        e_lo = w * EW
        e_hi = e_lo + EW
        zi = jnp.zeros((16,), jnp.int32)
        ones = jnp.ones((16,), jnp.int32)

        def zbody(i, _):
            hist_v[pl.ds(i * 16, 16)] = zi
            return 0

        lax.fori_loop(0, NPAD // 16, zbody, 0)

        def chunk_body(j, _):
            e0 = e_lo + j * K3
            pltpu.sync_copy(row_pad.at[pl.ds(e0, K3)], rw_v)
            for kk in range(K3 // 16):
                rowv = rw_v[pl.ds(kk * 16, 16)]
                ev = e0 + kk * 16 + _IOTA(16)
                plsc.addupdate_scatter(hist_v, [rowv], ones, mask=ev < e_hi)
            return 0

        lax.fori_loop(0, NCH3, chunk_body, 0)
        pltpu.sync_copy(hist_v, hist_o.at[w])

    return hist


def _make_sort():
    @functools.partial(
        pl.kernel,
        out_type=(jax.ShapeDtypeStruct((EPAD,), jnp.int32),
                  jax.ShapeDtypeStruct((EPAD,), jnp.int32),
                  jax.ShapeDtypeStruct((RP_PAD,), jnp.int32)),
        mesh=_MESH,
        compiler_params=_SC_PARAMS,
        scratch_types=[
            pltpu.VMEM((NPAD,), jnp.int32),
            pltpu.VMEM((NPAD,), jnp.int32),
            pltpu.VMEM((NPAD,), jnp.int32),
            pltpu.VMEM((K3,), jnp.int32),
            pltpu.VMEM((K3,), jnp.int32),
            pltpu.VMEM((K3,), jnp.int32),
            pltpu.VMEM((16,), jnp.int32),
            pltpu.SemaphoreType.DMA,
            pltpu.SemaphoreType.DMA,
        ],
        name="gnn_sort",
    )
    def csort(shist, row_pad, col_pad, srow_o, scol_o, rp_o,
              tmp_v, pre_v, tot_v, rw_v, cl_v, pos_v, c16_v, sem_s, sem_c):
        w = _wid()
        lo = w * NR
        e_lo = w * EW
        e_hi = e_lo + EW
        zi = jnp.zeros((16,), jnp.int32)
        ones = jnp.ones((16,), jnp.int32)
        iota = _IOTA(16)

        def zbody(i, _):
            pre_v[pl.ds(i * 16, 16)] = zi
            tot_v[pl.ds(i * 16, 16)] = zi
            return 0

        lax.fori_loop(0, NPAD // 16, zbody, 0)

        for wp in range(NW):
            pltpu.sync_copy(shist.at[wp], tmp_v)
            sel = jnp.where(wp < w, 1, 0)

            def abody(i, _):
                sl = pl.ds(i * 16, 16)
                t = tmp_v[sl]
                tot_v[sl] = tot_v[sl] + t
                pre_v[sl] = pre_v[sl] + t * sel
                return 0

            lax.fori_loop(0, NPAD // 16, abody, 0)

        def sbody(i, carry):
            sl = pl.ds(i * 16, 16)
            seg = tot_v[sl]
            c = plsc.cumsum(seg)
            tot_v[sl] = c - seg + carry
            return carry + c[15]

        lax.fori_loop(0, NPAD // 16, sbody, jnp.int32(0))

        def obody(i, _):
            sl = pl.ds(i * 16, 16)
            pre_v[sl] = pre_v[sl] + tot_v[sl]
            return 0

        lax.fori_loop(0, NPAD // 16, obody, 0)

        pltpu.sync_copy(tot_v.at[pl.ds(lo, NR)], rp_o.at[pl.ds(lo, NR)])

        @pl.when(w == 0)
        def _():
            c16_v[pl.ds(0, 16)] = jnp.full((16,), E, jnp.int32)
            pltpu.sync_copy(c16_v, rp_o.at[pl.ds(NPAD, 16)])
            for q in range(4):
                pos_v[pl.ds(q * 16, 16)] = zi
            pltpu.sync_copy(pos_v, srow_o.at[pl.ds(E, K3)])
            pltpu.sync_copy(pos_v, scol_o.at[pl.ds(E, K3)])

        def chunk_body(j, _):
            e0 = e_lo + j * K3
            pltpu.sync_copy(row_pad.at[pl.ds(e0, K3)], rw_v)
            pltpu.sync_copy(col_pad.at[pl.ds(e0, K3)], cl_v)
            for kk in range(K3 // 16):
                sl = pl.ds(kk * 16, 16)
                rowv = rw_v[sl]
                rank = zi
                for s in range(1, 16):
                    pm = iota >= s
                    prv = plsc.load_gather(rw_v, [kk * 16 + iota - s],
                                           mask=pm)
                    rank = rank + jnp.where(pm & (prv == rowv), 1, 0)
                base = plsc.load_gather(pre_v, [rowv])
                ev = e0 + kk * 16 + iota
                valid = ev < e_hi
                dump = E + K3 + w * 16 + iota
                pos_v[sl] = jnp.where(valid, base + rank, dump)
                plsc.addupdate_scatter(pre_v, [rowv], ones, mask=valid)
            cp_s = pltpu.async_copy(rw_v, srow_o.at[pos_v], sem_s)
            cp_c = pltpu.async_copy(cl_v, scol_o.at[pos_v], sem_c)
            cp_s.wait()
            cp_c.wait()
            return 0

        lax.fori_loop(0, NCH3, chunk_body, 0)

    return csort


def _make_ybuild():
    @functools.partial(
        pl.kernel,
        out_type=jax.ShapeDtypeStruct((NPAD, 128), jnp.float32),
        mesh=_MESH,
        compiler_params=_SC_PARAMS,
        scratch_types=[
            pltpu.VMEM((1024,), jnp.int32),
            pltpu.VMEM((NR + 16,), jnp.float32),
            pltpu.VMEM((NR, 128), jnp.float32),
        ],
        name="gnn_ybuild",
    )
    def ybuild(y_pad, roots, y_o, roots_v, fac_v, y_v):
        w = _wid()
        lo = w * NR
        hi = lo + NR
        pltpu.sync_copy(roots, roots_v)
        pltpu.sync_copy(y_pad.at[pl.ds(lo, NR)], y_v)
        ones = jnp.ones((16,), jnp.float32)
        zeros = jnp.zeros((16,), jnp.float32)
        for i in range((NR + 16) // 16):
            fac_v[pl.ds(i * 16, 16)] = ones
        for kk in range(1024 // 16):
            rv = roots_v[pl.ds(kk * 16, 16)]
            m = (rv >= lo) & (rv < hi)
            plsc.store_scatter(fac_v, [rv - lo], zeros, mask=m)

        def row_body(r, _):
            f = fac_v[pl.ds(r, 16)][0]
            for d in range(128 // 16):
                sl = pl.ds(d * 16, 16)
                y_v[r, sl] = y_v[r, sl] * f
            return 0

        lax.fori_loop(0, NR, row_body, 0)
        pltpu.sync_copy(y_v, y_o.at[pl.ds(lo, NR)])

    return ybuild


def _make_final_gather(dims):
    R = 1024
    RPW = R // NW

    @functools.partial(
        pl.kernel,
        out_type=tuple(jax.ShapeDtypeStruct((R, d), jnp.float32)
                       for d in dims),
        mesh=_MESH,
        compiler_params=_SC_PARAMS,
        scratch_types=[
            pltpu.VMEM((RPW,), jnp.int32),
            pltpu.VMEM((RPW, 256), jnp.float32),
            pltpu.VMEM((RPW, 128), jnp.float32),
            pltpu.SemaphoreType.DMA,
        ],
        name="gnn_final_gather",
    )
    def fgather(*args):
        srcs = args[:len(dims)]
        roots = args[len(dims)]
        outs = args[len(dims) + 1:len(dims) + 1 + len(dims)]
        ids_v, big_v, small_v, sem = args[len(dims) + 1 + len(dims):]
        w = _wid()
        r0 = w * RPW
        pltpu.sync_copy(roots.at[pl.ds(r0, RPW)], ids_v)
        for s, d, o in zip(srcs, dims, outs):
            buf = big_v if d == 256 else small_v
            pltpu.async_copy(s.at[ids_v], buf, sem).wait()
            pltpu.sync_copy(buf, o.at[pl.ds(r0, RPW)])

    return fgather


def _tc_matmul(parts, ws):
    n = len(parts)

    def body(*refs):
        o_ref = refs[-1]
        acc = jnp.zeros(o_ref.shape, jnp.float32)
        for a_ref, w_ref in zip(refs[:n], refs[n:-1]):
            acc = acc + jnp.dot(a_ref[...], w_ref[...],
                                preferred_element_type=jnp.float32)
        o_ref[...] = acc

    return pl.pallas_call(
        body,
        out_shape=jax.ShapeDtypeStruct((parts[0].shape[0], ws[0].shape[1]),
                                       jnp.float32),
    )(*parts, *ws)


def kernel(x, y_one_hot, W, sigmas, row, col, root_n_id):
    row_pad = jnp.pad(row, (0, EPAD - E))
    col_pad = jnp.pad(col, (0, EPAD - E))
    shist = _make_hist()(row_pad)
    srow, scol, row_ptr = _make_sort()(shist, row_pad, col_pad)
    sig16 = jnp.pad(sigmas, (0, 16 - sigmas.shape[0]))
    y_pad = jnp.pad(y_one_hot, ((0, NPAD - N), (0, 128 - 64)))

    feats = [x]
    feat = x
    for i in range(3):
        val, dis = _make_sweep1(256, i)(feat, srow, scol, row_ptr, sig16)
        feat = _make_sweep2(256)(feat, srow, scol, row_ptr, val, dis)
        feats.append(feat)

    g = _make_ybuild()(y_pad, root_n_id)
    for i in range(2):
        val, dis = _make_sweep1(128, 3 + i)(g, srow, scol, row_ptr, sig16)
        g = _make_sweep2(128)(g, srow, scol, row_ptr, val, dis)
        feats.append(g)

    parts = _make_final_gather((256, 256, 256, 256, 128, 128))(
        *feats, root_n_id)
    ws = [W[i * 256:(i + 1) * 256] for i in range(4)]
    ws.append(jnp.pad(W[1024:1088], ((0, 64), (0, 0))))
    ws.append(jnp.pad(W[1088:1152], ((0, 64), (0, 0))))
    return _tc_matmul(parts, ws)
